# baseline (speedup 1.0000x reference)
import jax
import jax.numpy as jnp
from jax.experimental import pallas as pl
from jax.experimental.pallas import tpu as pltpu

TAU = 0.5


def _body(logits_ref, uniform_ref, out_ref):
    u = uniform_ref[0]
    l = logits_ref[0]
    gumbel = -jnp.log(-jnp.log(u))
    x = (gumbel + l) * (1.0 / TAU)
    m = jnp.max(x, axis=1, keepdims=True)
    s = jnp.sum(jnp.exp(x - m), axis=1, keepdims=True)
    lse = m + jnp.log(s)
    y = jnp.max(x - lse, axis=0, keepdims=True)
    out_ref[0] = jnp.exp(y)


def kernel(logits, uniform):
    B, _, D = logits.shape
    K = uniform.shape[1]
    out = pl.pallas_call(
        _body,
        grid=(B,),
        in_specs=[
            pl.BlockSpec((1, 1, D), lambda b: (b, 0, 0)),
            pl.BlockSpec((1, K, D), lambda b: (b, 0, 0)),
        ],
        out_specs=pl.BlockSpec((1, 1, D), lambda b: (b, 0, 0)),
        out_shape=jax.ShapeDtypeStruct((B, 1, D), jnp.float32),
    )(logits, uniform)
    return out


# speedup vs baseline: 1.0027x; 1.0027x over previous
import functools

import jax
import jax.numpy as jnp
from jax.experimental import pallas as pl
from jax.experimental.pallas import tpu as pltpu

_C1 = 2.0 * 1.4426950408889634


def _body(logits_ref, uniform_ref, out_ref, p_ref):
    k = pl.program_id(1)

    @pl.when(k == 0)
    def _():
        p_ref[...] = jnp.exp2(_C1 * logits_ref[...])

    u = uniform_ref[...]
    l2u = jnp.log2(u)
    t = l2u * l2u
    e = p_ref[...] / t
    s = jnp.sum(e, axis=(2, 3), keepdims=True)
    y = e * (1.0 / s)

    @pl.when(k == 0)
    def _():
        out_ref[...] = y

    @pl.when(k > 0)
    def _():
        out_ref[...] = jnp.maximum(out_ref[...], y)


def kernel(logits, uniform):
    B, _, D = logits.shape
    K = uniform.shape[1]
    NB = 8
    DS = D // 128
    l4 = logits.reshape(B, 1, DS, 128)
    u4 = uniform.reshape(B, K, DS, 128)
    out = pl.pallas_call(
        _body,
        grid=(B // NB, K),
        in_specs=[
            pl.BlockSpec((NB, 1, DS, 128), lambda b, k: (b, 0, 0, 0)),
            pl.BlockSpec((NB, 1, DS, 128), lambda b, k: (b, k, 0, 0)),
        ],
        out_specs=pl.BlockSpec((NB, 1, DS, 128), lambda b, k: (b, 0, 0, 0)),
        out_shape=jax.ShapeDtypeStruct((B, 1, DS, 128), jnp.float32),
        scratch_shapes=[pltpu.VMEM((NB, 1, DS, 128), jnp.float32)],
        compiler_params=pltpu.CompilerParams(
            dimension_semantics=("arbitrary", "arbitrary"),
        ),
    )(l4, u4)
    return out.reshape(B, 1, D)
